# baseline (speedup 1.0000x reference)
import functools

import jax
import jax.numpy as jnp
from jax import lax
from jax.experimental import pallas as pl
from jax.experimental.pallas import tpu as pltpu
from jax.experimental.pallas import tpu_sc as plsc

NC = 2
NS = 16
L = 16
NW = NC * NS
KROW = 2
CHUNK = KROW * 128
ZCH = 64
ACC_PAD = 128


def _mesh():
    return plsc.VectorSubcoreMesh(core_axis_name="c", subcore_axis_name="s",
                                  num_cores=NC, num_subcores=NS)


def _sc_pass(N, E2, n_g, tasks, edge_split, gather):
    NA = N + ACC_PAD
    n_out = len(tasks)

    def _split8(total):
        main = -(-total // (NS * 8)) * 8
        last = total - (NS - 1) * main
        assert 0 < last <= main and main % 8 == 0
        return main, last

    fmain, flast = _split8(N)
    zmain, zlast = _split8(NA)
    rows_pt = (E2 // NW) if edge_split else (E2 // NS)
    nchunks = rows_pt // KROW
    assert nchunks % 2 == 0
    npairs = nchunks // 2

    def body(*refs):
        if gather:
            il = refs[0]
            g_refs = refs[1:1 + n_g]
            k0 = 1 + n_g
        else:
            il = refs[0]
            k0 = 1
        outs = refs[k0:k0 + n_out]
        if gather:
            acc, idx0, idx1, rows0, rows1, zbuf, semg, sems = refs[k0 + n_out:]
            idx = (idx0, idx1)
            rows = (rows0, rows1)
        else:
            acc, idx0, idx1, ones, zbuf, sems = refs[k0 + n_out:]
            idx = (idx0, idx1)

        c = lax.axis_index("c")
        s = lax.axis_index("s")

        zv = jnp.zeros((L,), jnp.float32)

        def fz(i, carry):
            zbuf[i, :] = zv
            return carry

        lax.fori_loop(0, ZCH, fz, 0)
        if not gather:
            ov = jnp.ones((L,), jnp.float32)

            def fo(i, carry):
                ones[i, :] = ov
                return carry

            lax.fori_loop(0, 128, fo, 0)

        def zero_block(r0, nrows):
            nf, rem = divmod(nrows, ZCH)
            zd = [pltpu.async_copy(zbuf, acc.at[pl.ds(r0 + k * ZCH, ZCH), :],
                                   sems)
                  for k in range(nf)]
            if rem:
                zd.append(pltpu.async_copy(
                    zbuf.at[pl.ds(0, rem), :],
                    acc.at[pl.ds(r0 + nf * ZCH, rem), :], sems))
            for d in zd:
                d.wait()

        def zero_acc():
            @pl.when(s < NS - 1)
            def _():
                zero_block(pl.multiple_of(s * zmain, 8), zmain)

            @pl.when(s == NS - 1)
            def _():
                zero_block((NS - 1) * zmain, zlast)

        def _maybe_when(cond, fn):
            if cond is True:
                fn()
            else:
                pl.when(cond)(fn)

        def run_task(g_ref, out_ref):
            plsc.subcore_barrier()
            zero_acc()
            plsc.subcore_barrier()
            if edge_split:
                base = (c * NS + s) * rows_pt
            else:
                base = s * rows_pt

            def copy_idx(i, b):
                r0 = pl.multiple_of((base + i * KROW) * 2, 4)
                pltpu.sync_copy(il.at[pl.ds(r0, 2 * KROW), :], idx[b])

            def fire_gather(b):
                for j in range(KROW):
                    pltpu.async_copy(g_ref.at[idx[b].at[j]],
                                     rows[b].at[pl.ds(j * 128, 128), :], semg)

            def drain_gather(b):
                for j in range(KROW):
                    pltpu.make_async_copy(
                        g_ref.at[idx[b].at[j]],
                        rows[b].at[pl.ds(j * 128, 128), :], semg).wait()

            def fire_scatter(b):
                for j in range(KROW):
                    src = rows[b].at[pl.ds(j * 128, 128), :] if gather else ones
                    pltpu.async_copy(src, acc.at[idx[b].at[KROW + j]], sems,
                                     add=True)

            def drain_scatter(b):
                for j in range(KROW):
                    src = rows[b].at[pl.ds(j * 128, 128), :] if gather else ones
                    pltpu.make_async_copy(src, acc.at[idx[b].at[KROW + j]],
                                          sems).wait()

            copy_idx(0, 0)
            if gather:
                fire_gather(0)

            def pair(p, carry):
                for b in (0, 1):
                    i = 2 * p + b
                    cond_prev = (p > 0) if b == 0 else True
                    cond_next = True if b == 0 else (p < npairs - 1)
                    nb = 1 - b
                    _maybe_when(cond_prev, lambda nb=nb: drain_scatter(nb))
                    _maybe_when(cond_next, lambda i=i, nb=nb: copy_idx(i + 1, nb))
                    if gather:
                        _maybe_when(cond_next, lambda nb=nb: fire_gather(nb))
                        drain_gather(b)
                    fire_scatter(b)
                return carry

            lax.fori_loop(0, npairs, pair, 0)
            drain_scatter(1)
            plsc.subcore_barrier()

            @pl.when(s < NS - 1)
            def _():
                r0 = pl.multiple_of(s * fmain, 8)
                pltpu.sync_copy(acc.at[pl.ds(r0, fmain), :],
                                out_ref.at[pl.ds(r0, fmain), :])

            @pl.when(s == NS - 1)
            def _():
                r0 = (NS - 1) * fmain
                pltpu.sync_copy(acc.at[pl.ds(r0, flast), :],
                                out_ref.at[pl.ds(r0, flast), :])

        for k, (core, gi) in enumerate(tasks):
            @pl.when(c == core)
            def _(k=k, gi=gi):
                run_task(g_refs[gi] if gather else None, outs[k])

    out_sds = tuple(jax.ShapeDtypeStruct((N, L), jnp.float32)
                    for _ in range(n_out))
    if gather:
        scratch = (
            [pltpu.VMEM_SHARED((NA, L), jnp.float32)]
            + [pltpu.VMEM((2 * KROW, 128), jnp.int32)] * 2
            + [pltpu.VMEM((CHUNK, L), jnp.float32)] * 2
            + [pltpu.VMEM((ZCH, L), jnp.float32),
               pltpu.SemaphoreType.DMA,
               pltpu.SemaphoreType.DMA]
        )
    else:
        scratch = (
            [pltpu.VMEM_SHARED((NA, L), jnp.float32)]
            + [pltpu.VMEM((2 * KROW, 128), jnp.int32)] * 2
            + [pltpu.VMEM((128, L), jnp.float32),
               pltpu.VMEM((ZCH, L), jnp.float32),
               pltpu.SemaphoreType.DMA]
        )
    return pl.kernel(body, out_type=out_sds, mesh=_mesh(),
                     scratch_types=scratch,
                     compiler_params=pltpu.CompilerParams(
                         use_tc_tiling_on_sc=False))


NBR = 25
BR = 500


def _kron_w(W, nci, nco):
    W4 = W.reshape(nci, 16, nco, 16)
    eye = jnp.eye(8, dtype=W.dtype)
    M6 = jnp.einsum("ikof,mn->imkonf", W4, eye)
    return M6.reshape(nci * 128, nco * 128)


def _tile_b(b):
    nco = b.shape[0] // 16
    return jnp.tile(b.reshape(nco, 1, 16), (1, 8, 1)).reshape(1, nco * 128)


def _row_spec():
    return pl.BlockSpec((None, BR, 128), lambda i: (i, 0, 0))


def _full_spec(shape):
    return pl.BlockSpec(shape, lambda i: tuple(0 for _ in shape))


def _tc_prep_body(d0, d1, xp, dinv, g1):
    deg = d0[...] + d1[...] + 1.0
    di = lax.rsqrt(deg)
    dinv[...] = di
    g1[...] = di * xp[...]


def _tc_layer1_body(s0, s1, g1, dinv, M1, b1r, o0, o1, o2, o3):
    di = dinv[...]
    p = di * (s0[...] + s1[...] + g1[...])
    h = jnp.maximum(
        jnp.dot(p, M1[...], preferred_element_type=jnp.float32) + b1r[...],
        0.0)
    g2 = jnp.concatenate([di] * 4, axis=1) * h
    o0[...] = g2[:, 0:128]
    o1[...] = g2[:, 128:256]
    o2[...] = g2[:, 256:384]
    o3[...] = g2[:, 384:512]


def _tc_layer2_body(s0, s1, s2, s3, g0, g1, g2, g3, dinv, M2, b2r, M3,
                    o0, o1):
    di = dinv[...]
    p = jnp.concatenate([di * (s0[...] + g0[...]), di * (s1[...] + g1[...]),
                         di * (s2[...] + g2[...]), di * (s3[...] + g3[...])],
                        axis=1)
    h = jnp.maximum(
        jnp.dot(p, M2[...], preferred_element_type=jnp.float32) + b2r[...],
        0.0)
    m = jnp.dot(h, M3[...], preferred_element_type=jnp.float32)
    go = jnp.concatenate([di, di], axis=1) * m
    o0[...] = go[:, 0:128]
    o1[...] = go[:, 128:256]


def _tc_final_body(nblk, n_nodes, s0, s1, g0, g1, dinv, b3r, out, accum):
    di = dinv[...]
    b3v = b3r[...]
    v0 = di * (s0[...] + g0[...]) + b3v[:, 0:128]
    v1 = di * (s1[...] + g1[...]) + b3v[:, 128:256]
    cur = jnp.concatenate(
        [jnp.sum(v0, axis=0, keepdims=True),
         jnp.sum(v1, axis=0, keepdims=True),
         jnp.max(v0, axis=0, keepdims=True),
         jnp.max(v1, axis=0, keepdims=True)], axis=1)
    i = pl.program_id(0)

    @pl.when(i == 0)
    def _():
        accum[...] = cur

    @pl.when(i > 0)
    def _():
        prev = accum[...]
        accum[...] = jnp.concatenate(
            [prev[:, 0:256] + cur[:, 0:256],
             jnp.maximum(prev[:, 256:512], cur[:, 256:512])], axis=1)

    @pl.when(i == nblk - 1)
    def _():
        a = accum[...]

        def fold(col0, op):
            r = a[:, col0:col0 + 16]
            for m in range(1, 8):
                r = op(r, a[:, col0 + 16 * m:col0 + 16 * m + 16])
            return r

        out[...] = jnp.concatenate(
            [fold(0, jnp.add) * (1.0 / n_nodes),
             fold(128, jnp.add) * (1.0 / n_nodes),
             fold(256, jnp.maximum), fold(384, jnp.maximum)], axis=1)


def kernel(x, edge_index, batch, W1, b1, W2, b2, W3, b3):
    N = x.shape[0]
    E = edge_index.shape[1]
    f32 = jnp.float32

    E_pad = -(-E // (2 * NW * CHUNK)) * (2 * NW * CHUNK)
    pad = E_pad - E
    src_p = jnp.concatenate([edge_index[0], jnp.zeros((pad,), jnp.int32)])
    dst_p = jnp.concatenate([edge_index[1], jnp.full((pad,), N, jnp.int32)])
    E2 = E_pad // 128
    src3 = src_p.reshape(E2 // KROW, KROW, 128)
    dst3 = dst_p.reshape(E2 // KROW, KROW, 128)
    il = jnp.concatenate([src3, dst3], axis=1).reshape(2 * E2, 128)

    NR = N * L // 128
    assert NR == NBR * BR
    nblk = NBR
    grid = (nblk,)
    sds_r = jax.ShapeDtypeStruct((NBR, BR, 128), f32)

    def r128(a):
        return a.reshape(NBR, BR, 128)

    def r16(a):
        return a.reshape(N, L)

    xp = jnp.pad(x, ((0, 0), (0, L - 4))).reshape(NBR, BR, 128)
    M1 = _kron_w(jnp.pad(W1, ((0, 12), (0, 0))), 1, 4)
    M2 = _kron_w(W2, 4, 4)
    M3 = _kron_w(W3, 4, 2)

    deg_fn = _sc_pass(N, E2, 0, [(0, None), (1, None)], True, False)
    degp0, degp1 = deg_fn(il)

    dinv, g1 = pl.pallas_call(
        _tc_prep_body,
        grid=grid,
        in_specs=[_row_spec()] * 3,
        out_specs=[_row_spec()] * 2,
        out_shape=[sds_r, sds_r],
    )(r128(degp0), r128(degp1), xp)

    l1_fn = _sc_pass(N, E2, 1, [(0, 0), (1, 0)], True, True)
    s1_0, s1_1 = l1_fn(il, r16(g1))

    g2 = pl.pallas_call(
        _tc_layer1_body,
        grid=grid,
        in_specs=[_row_spec()] * 4 + [_full_spec((128, 512)),
                                      _full_spec((1, 512))],
        out_specs=[_row_spec()] * 4,
        out_shape=[sds_r] * 4,
    )(r128(s1_0), r128(s1_1), g1, dinv, M1, _tile_b(b1))

    l2_fn = _sc_pass(N, E2, 4, [(0, 0), (0, 1), (1, 2), (1, 3)], False, True)
    s2 = l2_fn(il, *[r16(g) for g in g2])

    g3 = pl.pallas_call(
        _tc_layer2_body,
        grid=grid,
        in_specs=[_row_spec()] * 9 + [_full_spec((512, 512)),
                                      _full_spec((1, 512)),
                                      _full_spec((512, 256))],
        out_specs=[_row_spec()] * 2,
        out_shape=[sds_r] * 2,
    )(*[r128(s) for s in s2], *g2, dinv, M2, _tile_b(b2), M3)

    l3_fn = _sc_pass(N, E2, 2, [(0, 0), (1, 1)], False, True)
    s3 = l3_fn(il, r16(g3[0]), r16(g3[1]))

    out = pl.pallas_call(
        functools.partial(_tc_final_body, nblk, float(N)),
        grid=grid,
        in_specs=[_row_spec()] * 5 + [_full_spec((1, 256))],
        out_specs=pl.BlockSpec((1, 64), lambda i: (0, 0)),
        out_shape=jax.ShapeDtypeStruct((1, 64), f32),
        scratch_shapes=[pltpu.VMEM((1, 512), f32)],
    )(r128(s3[0]), r128(s3[1]), *g3, dinv, _tile_b(b3))
    return out


# speedup vs baseline: 1.3961x; 1.3961x over previous
import functools

import jax
import jax.numpy as jnp
from jax import lax
from jax.experimental import pallas as pl
from jax.experimental.pallas import tpu as pltpu
from jax.experimental.pallas import tpu_sc as plsc

NC = 2
NS = 16
L = 16
NW = NC * NS
KROW = 4
CHUNK = KROW * 128
ZCH = 64
ACC_PAD = 128


def _mesh():
    return plsc.VectorSubcoreMesh(core_axis_name="c", subcore_axis_name="s",
                                  num_cores=NC, num_subcores=NS)


def _sc_pass(N, E2, n_g, tasks, edge_split, gather):
    NA = N + ACC_PAD
    n_out = len(tasks)

    def _split8(total):
        main = -(-total // (NS * 8)) * 8
        last = total - (NS - 1) * main
        assert 0 < last <= main and main % 8 == 0
        return main, last

    fmain, flast = _split8(N)
    zmain, zlast = _split8(NA)
    rows_pt = (E2 // NW) if edge_split else (E2 // NS)
    nchunks = rows_pt // KROW
    assert nchunks % 2 == 0
    npairs = nchunks // 2

    def body(*refs):
        if gather:
            il = refs[0]
            g_refs = refs[1:1 + n_g]
            k0 = 1 + n_g
        else:
            il = refs[0]
            k0 = 1
        outs = refs[k0:k0 + n_out]
        if gather:
            acc, idx0, idx1, rows0, rows1, zbuf, semg, sems = refs[k0 + n_out:]
            idx = (idx0, idx1)
            rows = (rows0, rows1)
        else:
            acc, idx0, idx1, ones, zbuf, sems = refs[k0 + n_out:]
            idx = (idx0, idx1)

        c = lax.axis_index("c")
        s = lax.axis_index("s")

        zv = jnp.zeros((L,), jnp.float32)

        def fz(i, carry):
            zbuf[i, :] = zv
            return carry

        lax.fori_loop(0, ZCH, fz, 0)
        if not gather:
            ov = jnp.ones((L,), jnp.float32)

            def fo(i, carry):
                ones[i, :] = ov
                return carry

            lax.fori_loop(0, 128, fo, 0)

        def zero_block(r0, nrows):
            nf, rem = divmod(nrows, ZCH)
            zd = [pltpu.async_copy(zbuf, acc.at[pl.ds(r0 + k * ZCH, ZCH), :],
                                   sems)
                  for k in range(nf)]
            if rem:
                zd.append(pltpu.async_copy(
                    zbuf.at[pl.ds(0, rem), :],
                    acc.at[pl.ds(r0 + nf * ZCH, rem), :], sems))
            for d in zd:
                d.wait()

        def zero_acc():
            @pl.when(s < NS - 1)
            def _():
                zero_block(pl.multiple_of(s * zmain, 8), zmain)

            @pl.when(s == NS - 1)
            def _():
                zero_block((NS - 1) * zmain, zlast)

        def _maybe_when(cond, fn):
            if cond is True:
                fn()
            else:
                pl.when(cond)(fn)

        def run_task(g_ref, out_ref):
            plsc.subcore_barrier()
            zero_acc()
            plsc.subcore_barrier()
            if edge_split:
                base = (c * NS + s) * rows_pt
            else:
                base = s * rows_pt

            def copy_idx(i, b):
                r0 = pl.multiple_of((base + i * KROW) * 2, 8)
                pltpu.sync_copy(il.at[pl.ds(r0, 2 * KROW), :], idx[b])

            def fire_gather(b):
                for j in range(KROW):
                    pltpu.async_copy(g_ref.at[idx[b].at[j]],
                                     rows[b].at[pl.ds(j * 128, 128), :], semg)

            def drain_gather(b):
                for j in range(KROW):
                    pltpu.make_async_copy(
                        g_ref.at[idx[b].at[j]],
                        rows[b].at[pl.ds(j * 128, 128), :], semg).wait()

            def fire_scatter(b):
                for j in range(KROW):
                    src = rows[b].at[pl.ds(j * 128, 128), :] if gather else ones
                    pltpu.async_copy(src, acc.at[idx[b].at[KROW + j]], sems,
                                     add=True)

            def drain_scatter(b):
                for j in range(KROW):
                    src = rows[b].at[pl.ds(j * 128, 128), :] if gather else ones
                    pltpu.make_async_copy(src, acc.at[idx[b].at[KROW + j]],
                                          sems).wait()

            copy_idx(0, 0)
            if gather:
                fire_gather(0)

            def pair(p, carry):
                for b in (0, 1):
                    i = 2 * p + b
                    cond_prev = (p > 0) if b == 0 else True
                    cond_next = True if b == 0 else (p < npairs - 1)
                    nb = 1 - b
                    _maybe_when(cond_prev, lambda nb=nb: drain_scatter(nb))
                    _maybe_when(cond_next, lambda i=i, nb=nb: copy_idx(i + 1, nb))
                    if gather:
                        _maybe_when(cond_next, lambda nb=nb: fire_gather(nb))
                        drain_gather(b)
                    fire_scatter(b)
                return carry

            lax.fori_loop(0, npairs, pair, 0)
            drain_scatter(1)
            plsc.subcore_barrier()

            @pl.when(s < NS - 1)
            def _():
                r0 = pl.multiple_of(s * fmain, 8)
                pltpu.sync_copy(acc.at[pl.ds(r0, fmain), :],
                                out_ref.at[pl.ds(r0, fmain), :])

            @pl.when(s == NS - 1)
            def _():
                r0 = (NS - 1) * fmain
                pltpu.sync_copy(acc.at[pl.ds(r0, flast), :],
                                out_ref.at[pl.ds(r0, flast), :])

        for k, (core, gi) in enumerate(tasks):
            @pl.when(c == core)
            def _(k=k, gi=gi):
                run_task(g_refs[gi] if gather else None, outs[k])

    out_sds = tuple(jax.ShapeDtypeStruct((N, L), jnp.float32)
                    for _ in range(n_out))
    if gather:
        scratch = (
            [pltpu.VMEM_SHARED((NA, L), jnp.float32)]
            + [pltpu.VMEM((2 * KROW, 128), jnp.int32)] * 2
            + [pltpu.VMEM((CHUNK, L), jnp.float32)] * 2
            + [pltpu.VMEM((ZCH, L), jnp.float32),
               pltpu.SemaphoreType.DMA,
               pltpu.SemaphoreType.DMA]
        )
    else:
        scratch = (
            [pltpu.VMEM_SHARED((NA, L), jnp.float32)]
            + [pltpu.VMEM((2 * KROW, 128), jnp.int32)] * 2
            + [pltpu.VMEM((128, L), jnp.float32),
               pltpu.VMEM((ZCH, L), jnp.float32),
               pltpu.SemaphoreType.DMA]
        )
    return pl.kernel(body, out_type=out_sds, mesh=_mesh(),
                     scratch_types=scratch,
                     compiler_params=pltpu.CompilerParams(
                         use_tc_tiling_on_sc=False))


NBR = 10
BR = 1250


def _kron_w(W, nci, nco):
    W4 = W.reshape(nci, 16, nco, 16)
    eye = jnp.eye(8, dtype=W.dtype)
    M6 = jnp.einsum("ikof,mn->imkonf", W4, eye)
    return M6.reshape(nci * 128, nco * 128)


def _tile_b(b):
    nco = b.shape[0] // 16
    return jnp.tile(b.reshape(nco, 1, 16), (1, 8, 1)).reshape(1, nco * 128)


def _row_spec():
    return pl.BlockSpec((None, BR, 128), lambda i: (i, 0, 0))


def _full_spec(shape):
    return pl.BlockSpec(shape, lambda i: tuple(0 for _ in shape))


def _tc_prep_body(d0, d1, xp, dinv, g1):
    deg = d0[...] + d1[...] + 1.0
    di = lax.rsqrt(deg)
    dinv[...] = di
    g1[...] = di * xp[...]


def _tc_layer1_body(s0, s1, g1, dinv, M1, b1r, o0, o1, o2, o3):
    di = dinv[...]
    p = di * (s0[...] + s1[...] + g1[...])
    h = jnp.maximum(
        jnp.dot(p, M1[...], preferred_element_type=jnp.float32) + b1r[...],
        0.0)
    g2 = jnp.concatenate([di] * 4, axis=1) * h
    o0[...] = g2[:, 0:128]
    o1[...] = g2[:, 128:256]
    o2[...] = g2[:, 256:384]
    o3[...] = g2[:, 384:512]


def _tc_layer2_body(s0, s1, s2, s3, g0, g1, g2, g3, dinv, M2, b2r, M3,
                    o0, o1):
    di = dinv[...]
    p = jnp.concatenate([di * (s0[...] + g0[...]), di * (s1[...] + g1[...]),
                         di * (s2[...] + g2[...]), di * (s3[...] + g3[...])],
                        axis=1)
    h = jnp.maximum(
        jnp.dot(p, M2[...], preferred_element_type=jnp.float32) + b2r[...],
        0.0)
    m = jnp.dot(h, M3[...], preferred_element_type=jnp.float32)
    go = jnp.concatenate([di, di], axis=1) * m
    o0[...] = go[:, 0:128]
    o1[...] = go[:, 128:256]


def _tc_final_body(nblk, n_nodes, s0, s1, g0, g1, dinv, b3r, out, accum):
    di = dinv[...]
    b3v = b3r[...]
    v0 = di * (s0[...] + g0[...]) + b3v[:, 0:128]
    v1 = di * (s1[...] + g1[...]) + b3v[:, 128:256]
    cur = jnp.concatenate(
        [jnp.sum(v0, axis=0, keepdims=True),
         jnp.sum(v1, axis=0, keepdims=True),
         jnp.max(v0, axis=0, keepdims=True),
         jnp.max(v1, axis=0, keepdims=True)], axis=1)
    i = pl.program_id(0)

    @pl.when(i == 0)
    def _():
        accum[...] = cur

    @pl.when(i > 0)
    def _():
        prev = accum[...]
        accum[...] = jnp.concatenate(
            [prev[:, 0:256] + cur[:, 0:256],
             jnp.maximum(prev[:, 256:512], cur[:, 256:512])], axis=1)

    @pl.when(i == nblk - 1)
    def _():
        a = accum[...]

        def fold(col0, op):
            r = a[:, col0:col0 + 16]
            for m in range(1, 8):
                r = op(r, a[:, col0 + 16 * m:col0 + 16 * m + 16])
            return r

        out[...] = jnp.concatenate(
            [fold(0, jnp.add) * (1.0 / n_nodes),
             fold(128, jnp.add) * (1.0 / n_nodes),
             fold(256, jnp.maximum), fold(384, jnp.maximum)], axis=1)


def kernel(x, edge_index, batch, W1, b1, W2, b2, W3, b3):
    N = x.shape[0]
    E = edge_index.shape[1]
    f32 = jnp.float32

    E_pad = -(-E // (2 * NW * CHUNK)) * (2 * NW * CHUNK)
    pad = E_pad - E
    src_p = jnp.concatenate([edge_index[0], jnp.zeros((pad,), jnp.int32)])
    dst_p = jnp.concatenate([edge_index[1], jnp.full((pad,), N, jnp.int32)])
    E2 = E_pad // 128
    src3 = src_p.reshape(E2 // KROW, KROW, 128)
    dst3 = dst_p.reshape(E2 // KROW, KROW, 128)
    il = jnp.concatenate([src3, dst3], axis=1).reshape(2 * E2, 128)

    NR = N * L // 128
    assert NR == NBR * BR
    nblk = NBR
    grid = (nblk,)
    sds_r = jax.ShapeDtypeStruct((NBR, BR, 128), f32)

    def r128(a):
        return a.reshape(NBR, BR, 128)

    def r16(a):
        return a.reshape(N, L)

    xp = jnp.pad(x, ((0, 0), (0, L - 4))).reshape(NBR, BR, 128)
    M1 = _kron_w(jnp.pad(W1, ((0, 12), (0, 0))), 1, 4)
    M2 = _kron_w(W2, 4, 4)
    M3 = _kron_w(W3, 4, 2)

    deg_fn = _sc_pass(N, E2, 0, [(0, None), (1, None)], True, False)
    degp0, degp1 = deg_fn(il)

    dinv, g1 = pl.pallas_call(
        _tc_prep_body,
        grid=grid,
        in_specs=[_row_spec()] * 3,
        out_specs=[_row_spec()] * 2,
        out_shape=[sds_r, sds_r],
    )(r128(degp0), r128(degp1), xp)

    l1_fn = _sc_pass(N, E2, 1, [(0, 0), (1, 0)], True, True)
    s1_0, s1_1 = l1_fn(il, r16(g1))

    g2 = pl.pallas_call(
        _tc_layer1_body,
        grid=grid,
        in_specs=[_row_spec()] * 4 + [_full_spec((128, 512)),
                                      _full_spec((1, 512))],
        out_specs=[_row_spec()] * 4,
        out_shape=[sds_r] * 4,
    )(r128(s1_0), r128(s1_1), g1, dinv, M1, _tile_b(b1))

    l2_fn = _sc_pass(N, E2, 4, [(0, 0), (0, 1), (1, 2), (1, 3)], False, True)
    s2 = l2_fn(il, *[r16(g) for g in g2])

    g3 = pl.pallas_call(
        _tc_layer2_body,
        grid=grid,
        in_specs=[_row_spec()] * 9 + [_full_spec((512, 512)),
                                      _full_spec((1, 512)),
                                      _full_spec((512, 256))],
        out_specs=[_row_spec()] * 2,
        out_shape=[sds_r] * 2,
    )(*[r128(s) for s in s2], *g2, dinv, M2, _tile_b(b2), M3)

    l3_fn = _sc_pass(N, E2, 2, [(0, 0), (1, 1)], False, True)
    s3 = l3_fn(il, r16(g3[0]), r16(g3[1]))

    out = pl.pallas_call(
        functools.partial(_tc_final_body, nblk, float(N)),
        grid=grid,
        in_specs=[_row_spec()] * 5 + [_full_spec((1, 256))],
        out_specs=pl.BlockSpec((1, 64), lambda i: (0, 0)),
        out_shape=jax.ShapeDtypeStruct((1, 64), f32),
        scratch_shapes=[pltpu.VMEM((1, 512), f32)],
    )(r128(s3[0]), r128(s3[1]), *g3, dinv, _tile_b(b3))
    return out


# speedup vs baseline: 1.4006x; 1.0032x over previous
import functools

import jax
import jax.numpy as jnp
from jax import lax
from jax.experimental import pallas as pl
from jax.experimental.pallas import tpu as pltpu
from jax.experimental.pallas import tpu_sc as plsc

NC = 2
NS = 16
L = 16
NW = NC * NS
KROW = 4
CHUNK = KROW * 128
ZCH = 64
ACC_PAD = 128


def _mesh():
    return plsc.VectorSubcoreMesh(core_axis_name="c", subcore_axis_name="s",
                                  num_cores=NC, num_subcores=NS)


def _sc_pass(N, E2, n_g, tasks, edge_split, gather):
    NA = N + ACC_PAD
    n_out = len(tasks)

    def _split8(total):
        main = -(-total // (NS * 8)) * 8
        last = total - (NS - 1) * main
        assert 0 < last <= main and main % 8 == 0
        return main, last

    fmain, flast = _split8(N)
    zmain, zlast = _split8(NA)
    rows_pt = (E2 // NW) if edge_split else (E2 // NS)
    nchunks = rows_pt // KROW
    assert nchunks % 2 == 0
    npairs = nchunks // 2

    def body(*refs):
        if gather:
            il = refs[0]
            g_refs = refs[1:1 + n_g]
            k0 = 1 + n_g
        else:
            il = refs[0]
            k0 = 1
        outs = refs[k0:k0 + n_out]
        if gather:
            acc, idx0, idx1, rows0, rows1, zbuf, semg, sems = refs[k0 + n_out:]
            idx = (idx0, idx1)
            rows = (rows0, rows1)
        else:
            acc, idx0, idx1, ones, zbuf, sems = refs[k0 + n_out:]
            idx = (idx0, idx1)

        c = lax.axis_index("c")
        s = lax.axis_index("s")

        zv = jnp.zeros((L,), jnp.float32)

        def fz(i, carry):
            zbuf[i, :] = zv
            return carry

        lax.fori_loop(0, ZCH, fz, 0)
        if not gather:
            ov = jnp.ones((L,), jnp.float32)

            def fo(i, carry):
                ones[i, :] = ov
                return carry

            lax.fori_loop(0, 128, fo, 0)

        def zero_block(r0, nrows):
            nf, rem = divmod(nrows, ZCH)
            zd = [pltpu.async_copy(zbuf, acc.at[pl.ds(r0 + k * ZCH, ZCH), :],
                                   sems)
                  for k in range(nf)]
            if rem:
                zd.append(pltpu.async_copy(
                    zbuf.at[pl.ds(0, rem), :],
                    acc.at[pl.ds(r0 + nf * ZCH, rem), :], sems))
            for d in zd:
                d.wait()

        def zero_acc():
            @pl.when(s < NS - 1)
            def _():
                zero_block(pl.multiple_of(s * zmain, 8), zmain)

            @pl.when(s == NS - 1)
            def _():
                zero_block((NS - 1) * zmain, zlast)

        def _maybe_when(cond, fn):
            if cond is True:
                fn()
            else:
                pl.when(cond)(fn)

        def run_task(g_ref, out_ref):
            plsc.subcore_barrier()
            zero_acc()
            plsc.subcore_barrier()
            if edge_split:
                base = (c * NS + s) * rows_pt
            else:
                base = s * rows_pt

            def copy_idx(i, b):
                r0 = pl.multiple_of((base + i * KROW) * 2, 8)
                if gather:
                    pltpu.sync_copy(il.at[pl.ds(r0, 2 * KROW), :], idx[b])
                else:
                    pltpu.sync_copy(
                        il.at[pl.ds(pl.multiple_of(r0 + KROW, 4), KROW), :],
                        idx[b])

            def fire_gather(b):
                for j in range(KROW):
                    pltpu.async_copy(g_ref.at[idx[b].at[j]],
                                     rows[b].at[pl.ds(j * 128, 128), :], semg)

            def drain_gather(b):
                for j in range(KROW):
                    pltpu.make_async_copy(
                        g_ref.at[idx[b].at[j]],
                        rows[b].at[pl.ds(j * 128, 128), :], semg).wait()

            dj = KROW if gather else 0

            def fire_scatter(b):
                for j in range(KROW):
                    src = rows[b].at[pl.ds(j * 128, 128), :] if gather else ones
                    pltpu.async_copy(src, acc.at[idx[b].at[dj + j]], sems,
                                     add=True)

            def drain_scatter(b):
                for j in range(KROW):
                    src = rows[b].at[pl.ds(j * 128, 128), :] if gather else ones
                    pltpu.make_async_copy(src, acc.at[idx[b].at[dj + j]],
                                          sems).wait()

            copy_idx(0, 0)
            if gather:
                fire_gather(0)

            def pair(p, carry):
                for b in (0, 1):
                    i = 2 * p + b
                    cond_prev = (p > 0) if b == 0 else True
                    cond_next = True if b == 0 else (p < npairs - 1)
                    nb = 1 - b
                    _maybe_when(cond_prev, lambda nb=nb: drain_scatter(nb))
                    _maybe_when(cond_next, lambda i=i, nb=nb: copy_idx(i + 1, nb))
                    if gather:
                        _maybe_when(cond_next, lambda nb=nb: fire_gather(nb))
                        drain_gather(b)
                    fire_scatter(b)
                return carry

            lax.fori_loop(0, npairs, pair, 0)
            drain_scatter(1)
            plsc.subcore_barrier()

            @pl.when(s < NS - 1)
            def _():
                r0 = pl.multiple_of(s * fmain, 8)
                pltpu.sync_copy(acc.at[pl.ds(r0, fmain), :],
                                out_ref.at[pl.ds(r0, fmain), :])

            @pl.when(s == NS - 1)
            def _():
                r0 = (NS - 1) * fmain
                pltpu.sync_copy(acc.at[pl.ds(r0, flast), :],
                                out_ref.at[pl.ds(r0, flast), :])

        for k, (core, gi) in enumerate(tasks):
            @pl.when(c == core)
            def _(k=k, gi=gi):
                run_task(g_refs[gi] if gather else None, outs[k])

    out_sds = tuple(jax.ShapeDtypeStruct((N, L), jnp.float32)
                    for _ in range(n_out))
    if gather:
        scratch = (
            [pltpu.VMEM_SHARED((NA, L), jnp.float32)]
            + [pltpu.VMEM((2 * KROW, 128), jnp.int32)] * 2
            + [pltpu.VMEM((CHUNK, L), jnp.float32)] * 2
            + [pltpu.VMEM((ZCH, L), jnp.float32),
               pltpu.SemaphoreType.DMA,
               pltpu.SemaphoreType.DMA]
        )
    else:
        scratch = (
            [pltpu.VMEM_SHARED((NA, L), jnp.float32)]
            + [pltpu.VMEM((KROW, 128), jnp.int32)] * 2
            + [pltpu.VMEM((128, L), jnp.float32),
               pltpu.VMEM((ZCH, L), jnp.float32),
               pltpu.SemaphoreType.DMA]
        )
    return pl.kernel(body, out_type=out_sds, mesh=_mesh(),
                     scratch_types=scratch,
                     compiler_params=pltpu.CompilerParams(
                         use_tc_tiling_on_sc=False))


NBR = 10
BR = 1250


def _kron_w(W, nci, nco):
    W4 = W.reshape(nci, 16, nco, 16)
    eye = jnp.eye(8, dtype=W.dtype)
    M6 = jnp.einsum("ikof,mn->imkonf", W4, eye)
    return M6.reshape(nci * 128, nco * 128)


def _tile_b(b):
    nco = b.shape[0] // 16
    return jnp.tile(b.reshape(nco, 1, 16), (1, 8, 1)).reshape(1, nco * 128)


def _row_spec():
    return pl.BlockSpec((None, BR, 128), lambda i: (i, 0, 0))


def _full_spec(shape):
    return pl.BlockSpec(shape, lambda i: tuple(0 for _ in shape))


def _tc_prep_body(d0, d1, xp, dinv, g1):
    deg = d0[...] + d1[...] + 1.0
    di = lax.rsqrt(deg)
    dinv[...] = di
    g1[...] = di * xp[...]


def _tc_layer1_body(s0, s1, g1, dinv, M1, b1r, o0, o1, o2, o3):
    di = dinv[...]
    p = di * (s0[...] + s1[...] + g1[...])
    h = jnp.maximum(
        jnp.dot(p, M1[...], preferred_element_type=jnp.float32) + b1r[...],
        0.0)
    g2 = jnp.concatenate([di] * 4, axis=1) * h
    o0[...] = g2[:, 0:128]
    o1[...] = g2[:, 128:256]
    o2[...] = g2[:, 256:384]
    o3[...] = g2[:, 384:512]


def _tc_layer2_body(s0, s1, s2, s3, g0, g1, g2, g3, dinv, M2, b2r, M3,
                    o0, o1):
    di = dinv[...]
    p = jnp.concatenate([di * (s0[...] + g0[...]), di * (s1[...] + g1[...]),
                         di * (s2[...] + g2[...]), di * (s3[...] + g3[...])],
                        axis=1)
    h = jnp.maximum(
        jnp.dot(p, M2[...], preferred_element_type=jnp.float32) + b2r[...],
        0.0)
    m = jnp.dot(h, M3[...], preferred_element_type=jnp.float32)
    go = jnp.concatenate([di, di], axis=1) * m
    o0[...] = go[:, 0:128]
    o1[...] = go[:, 128:256]


def _tc_final_body(nblk, n_nodes, s0, s1, g0, g1, dinv, b3r, out, accum):
    di = dinv[...]
    b3v = b3r[...]
    v0 = di * (s0[...] + g0[...]) + b3v[:, 0:128]
    v1 = di * (s1[...] + g1[...]) + b3v[:, 128:256]
    cur = jnp.concatenate(
        [jnp.sum(v0, axis=0, keepdims=True),
         jnp.sum(v1, axis=0, keepdims=True),
         jnp.max(v0, axis=0, keepdims=True),
         jnp.max(v1, axis=0, keepdims=True)], axis=1)
    i = pl.program_id(0)

    @pl.when(i == 0)
    def _():
        accum[...] = cur

    @pl.when(i > 0)
    def _():
        prev = accum[...]
        accum[...] = jnp.concatenate(
            [prev[:, 0:256] + cur[:, 0:256],
             jnp.maximum(prev[:, 256:512], cur[:, 256:512])], axis=1)

    @pl.when(i == nblk - 1)
    def _():
        a = accum[...]

        def fold(col0, op):
            r = a[:, col0:col0 + 16]
            for m in range(1, 8):
                r = op(r, a[:, col0 + 16 * m:col0 + 16 * m + 16])
            return r

        out[...] = jnp.concatenate(
            [fold(0, jnp.add) * (1.0 / n_nodes),
             fold(128, jnp.add) * (1.0 / n_nodes),
             fold(256, jnp.maximum), fold(384, jnp.maximum)], axis=1)


def kernel(x, edge_index, batch, W1, b1, W2, b2, W3, b3):
    N = x.shape[0]
    E = edge_index.shape[1]
    f32 = jnp.float32

    E_pad = -(-E // (2 * NW * CHUNK)) * (2 * NW * CHUNK)
    pad = E_pad - E
    src_p = jnp.concatenate([edge_index[0], jnp.zeros((pad,), jnp.int32)])
    dst_p = jnp.concatenate([edge_index[1], jnp.full((pad,), N, jnp.int32)])
    E2 = E_pad // 128
    src3 = src_p.reshape(E2 // KROW, KROW, 128)
    dst3 = dst_p.reshape(E2 // KROW, KROW, 128)
    il = jnp.concatenate([src3, dst3], axis=1).reshape(2 * E2, 128)

    NR = N * L // 128
    assert NR == NBR * BR
    nblk = NBR
    grid = (nblk,)
    sds_r = jax.ShapeDtypeStruct((NBR, BR, 128), f32)

    def r128(a):
        return a.reshape(NBR, BR, 128)

    def r16(a):
        return a.reshape(N, L)

    xp = jnp.pad(x, ((0, 0), (0, L - 4))).reshape(NBR, BR, 128)
    M1 = _kron_w(jnp.pad(W1, ((0, 12), (0, 0))), 1, 4)
    M2 = _kron_w(W2, 4, 4)
    M3 = _kron_w(W3, 4, 2)

    deg_fn = _sc_pass(N, E2, 0, [(0, None), (1, None)], True, False)
    degp0, degp1 = deg_fn(il)

    dinv, g1 = pl.pallas_call(
        _tc_prep_body,
        grid=grid,
        in_specs=[_row_spec()] * 3,
        out_specs=[_row_spec()] * 2,
        out_shape=[sds_r, sds_r],
    )(r128(degp0), r128(degp1), xp)

    l1_fn = _sc_pass(N, E2, 1, [(0, 0), (1, 0)], True, True)
    s1_0, s1_1 = l1_fn(il, r16(g1))

    g2 = pl.pallas_call(
        _tc_layer1_body,
        grid=grid,
        in_specs=[_row_spec()] * 4 + [_full_spec((128, 512)),
                                      _full_spec((1, 512))],
        out_specs=[_row_spec()] * 4,
        out_shape=[sds_r] * 4,
    )(r128(s1_0), r128(s1_1), g1, dinv, M1, _tile_b(b1))

    l2_fn = _sc_pass(N, E2, 4, [(0, 0), (0, 1), (1, 2), (1, 3)], False, True)
    s2 = l2_fn(il, *[r16(g) for g in g2])

    g3 = pl.pallas_call(
        _tc_layer2_body,
        grid=grid,
        in_specs=[_row_spec()] * 9 + [_full_spec((512, 512)),
                                      _full_spec((1, 512)),
                                      _full_spec((512, 256))],
        out_specs=[_row_spec()] * 2,
        out_shape=[sds_r] * 2,
    )(*[r128(s) for s in s2], *g2, dinv, M2, _tile_b(b2), M3)

    l3_fn = _sc_pass(N, E2, 2, [(0, 0), (1, 1)], False, True)
    s3 = l3_fn(il, r16(g3[0]), r16(g3[1]))

    out = pl.pallas_call(
        functools.partial(_tc_final_body, nblk, float(N)),
        grid=grid,
        in_specs=[_row_spec()] * 5 + [_full_spec((1, 256))],
        out_specs=pl.BlockSpec((1, 64), lambda i: (0, 0)),
        out_shape=jax.ShapeDtypeStruct((1, 64), f32),
        scratch_shapes=[pltpu.VMEM((1, 512), f32)],
    )(r128(s3[0]), r128(s3[1]), *g3, dinv, _tile_b(b3))
    return out
